# Initial kernel scaffold; baseline (speedup 1.0000x reference)
import jax
import jax.numpy as jnp
from jax.experimental import pallas as pl


def kernel(x, edge_index, W1, b1, Wm, bm, Wlv, blv):
    raise NotImplementedError("write your pallas kernel here")


# speedup vs baseline: 20.7141x; 20.7141x over previous
import functools

import jax
import jax.numpy as jnp
from jax import lax
from jax.experimental import pallas as pl
from jax.experimental.pallas import tpu as pltpu
from jax.experimental.pallas import tpu_sc as plsc

N = 10000
NPAD = 10240
E = 320000
IN_DIM = 128
HID = 256
LAT = 64

CHUNK = 128
NCHUNK = E // CHUNK
NW = 32
MAXIT = -(-NCHUNK // NW)
RPS = NPAD // 16

_BLK = 1000

_mesh = plsc.VectorSubcoreMesh(core_axis_name="c", subcore_axis_name="s")


@functools.partial(
    pl.kernel,
    out_type=jax.ShapeDtypeStruct((2, NPAD), jnp.float32),
    mesh=_mesh,
    scratch_types=[
        pltpu.VMEM((CHUNK,), jnp.int32),
        pltpu.VMEM((CHUNK,), jnp.float32),
        pltpu.VMEM_SHARED((NPAD,), jnp.float32),
    ],
)
def _deg_kernel(dst_hbm, zer_hbm, out_hbm, idx_d, ones_v, acc):
    c = lax.axis_index("c")
    s = lax.axis_index("s")
    w = s * 2 + c
    for k in range(CHUNK // 16):
        ones_v[pl.ds(k * 16, 16)] = jnp.ones((16,), jnp.float32)
    pltpu.sync_copy(zer_hbm.at[pl.ds(s * RPS, RPS)], acc.at[pl.ds(s * RPS, RPS)])
    plsc.subcore_barrier()

    def body(i, carry):
        row = w + i * NW

        @pl.when(row < NCHUNK)
        def _():
            pltpu.sync_copy(dst_hbm.at[pl.ds(row * CHUNK, CHUNK)], idx_d)
            pltpu.sync_copy(ones_v, acc.at[idx_d], add=True)

        return carry

    lax.fori_loop(0, MAXIT, body, 0)
    plsc.subcore_barrier()
    pltpu.sync_copy(acc.at[pl.ds(s * RPS, RPS)], out_hbm.at[c, pl.ds(s * RPS, RPS)])


@functools.partial(
    pl.kernel,
    out_type=jax.ShapeDtypeStruct((2, NPAD, IN_DIM), jnp.float32),
    mesh=_mesh,
    scratch_types=[
        pltpu.VMEM((CHUNK,), jnp.int32),
        pltpu.VMEM((CHUNK,), jnp.int32),
        pltpu.VMEM((CHUNK, IN_DIM), jnp.float32),
        pltpu.VMEM_SHARED((NPAD, IN_DIM), jnp.float32),
        pltpu.SemaphoreType.DMA,
    ],
)
def _agg_kernel(feat_hbm, src_hbm, dst_hbm, zer_hbm, out_hbm,
                idx_s, idx_d, rows, acc, sem):
    c = lax.axis_index("c")
    s = lax.axis_index("s")
    w = s * 2 + c
    pltpu.sync_copy(zer_hbm.at[pl.ds(s * RPS, RPS)], acc.at[pl.ds(s * RPS, RPS)])
    plsc.subcore_barrier()

    def body(i, carry):
        row = w + i * NW

        @pl.when(row < NCHUNK)
        def _():
            pltpu.sync_copy(src_hbm.at[pl.ds(row * CHUNK, CHUNK)], idx_s)
            pltpu.sync_copy(dst_hbm.at[pl.ds(row * CHUNK, CHUNK)], idx_d)
            pltpu.async_copy(feat_hbm.at[idx_s], rows, sem).wait()
            pltpu.sync_copy(rows, acc.at[idx_d], add=True)

        return carry

    lax.fori_loop(0, MAXIT, body, 0)
    plsc.subcore_barrier()
    pltpu.sync_copy(acc.at[pl.ds(s * RPS, RPS)],
                    out_hbm.at[c, pl.ds(s * RPS, RPS)])


def _tc1_body(deg_ref, x_ref, y1_ref):
    dinv = lax.rsqrt(deg_ref[0] + deg_ref[1] + 1.0)
    y1_ref[...] = x_ref[...] * dinv


def _tc2_body(deg_ref, agg_ref, y1_ref, w1_ref, b1_ref, wc_ref, y2_ref):
    dinv = lax.rsqrt(deg_ref[0] + deg_ref[1] + 1.0)
    t = (agg_ref[0] + agg_ref[1] + y1_ref[...]) * dinv
    h = jnp.dot(t, w1_ref[...], preferred_element_type=jnp.float32) + b1_ref[...]
    u = jnp.dot(h, wc_ref[...], preferred_element_type=jnp.float32)
    y2_ref[...] = u * dinv


def _tc3_body(deg_ref, agg_ref, y2_ref, bm_ref, blv_ref, noise_ref,
              z_ref, mean_ref, lv_ref):
    dinv = lax.rsqrt(deg_ref[0] + deg_ref[1] + 1.0)
    v = (agg_ref[0] + agg_ref[1] + y2_ref[...]) * dinv
    mean = v[:, :LAT] + bm_ref[...]
    lv = v[:, LAT:] + blv_ref[...]
    mean_ref[...] = mean
    lv_ref[...] = lv
    z_ref[...] = noise_ref[...] * jnp.exp(0.5 * lv) + mean


def _deg_spec():
    return pl.BlockSpec((2, _BLK, 1), lambda i: (0, i, 0))


def _agg_spec():
    return pl.BlockSpec((2, _BLK, IN_DIM), lambda i: (0, i, 0))


def _row_spec(d):
    return pl.BlockSpec((_BLK, d), lambda i: (i, 0))


def _full_spec(r, c):
    return pl.BlockSpec((r, c), lambda i: (0, 0))


def kernel(x, edge_index, W1, b1, Wm, bm, Wlv, blv):
    src = edge_index[0].astype(jnp.int32)
    dst = edge_index[1].astype(jnp.int32)
    zer1 = jnp.zeros((NPAD,), jnp.float32)
    zer2 = jnp.zeros((NPAD, IN_DIM), jnp.float32)
    Wcat = jnp.concatenate([Wm, Wlv], axis=1)
    noise = jax.random.normal(jax.random.key(42), (N, LAT), dtype=x.dtype)

    degp = _deg_kernel(dst, zer1)
    deg3 = degp.reshape(2, NPAD, 1)

    grid = (N // _BLK,)
    y1 = pl.pallas_call(
        _tc1_body,
        grid=grid,
        in_specs=[_deg_spec(), _row_spec(IN_DIM)],
        out_specs=_row_spec(IN_DIM),
        out_shape=jax.ShapeDtypeStruct((N, IN_DIM), jnp.float32),
    )(deg3, x)

    agg1 = _agg_kernel(y1, src, dst, zer2)

    y2 = pl.pallas_call(
        _tc2_body,
        grid=grid,
        in_specs=[_deg_spec(), _agg_spec(), _row_spec(IN_DIM),
                  _full_spec(IN_DIM, HID), _full_spec(1, HID),
                  _full_spec(HID, IN_DIM)],
        out_specs=_row_spec(IN_DIM),
        out_shape=jax.ShapeDtypeStruct((N, IN_DIM), jnp.float32),
    )(deg3, agg1, y1, W1, b1.reshape(1, HID), Wcat)

    agg2 = _agg_kernel(y2, src, dst, zer2)

    z, mean, lv = pl.pallas_call(
        _tc3_body,
        grid=grid,
        in_specs=[_deg_spec(), _agg_spec(), _row_spec(IN_DIM),
                  _full_spec(1, LAT), _full_spec(1, LAT), _row_spec(LAT)],
        out_specs=[_row_spec(LAT), _row_spec(LAT), _row_spec(LAT)],
        out_shape=[jax.ShapeDtypeStruct((N, LAT), jnp.float32)] * 3,
    )(deg3, agg2, y2, bm.reshape(1, LAT), blv.reshape(1, LAT), noise)

    return (z, mean, lv)


# speedup vs baseline: 29.5148x; 1.4249x over previous
import functools

import jax
import jax.numpy as jnp
from jax import lax
from jax.experimental import pallas as pl
from jax.experimental.pallas import tpu as pltpu
from jax.experimental.pallas import tpu_sc as plsc

N = 10000
NPAD = 10240
E = 320000
IN_DIM = 128
HID = 256
LAT = 64

CHUNK = 128
NW = 32
NSUP = -(-E // (CHUNK * NW))
NSUP += NSUP % 2
E_PAD = NSUP * CHUNK * NW
RPS = NPAD // 16

_BLK = 1000

_mesh = plsc.VectorSubcoreMesh(core_axis_name="c", subcore_axis_name="s")


@functools.partial(
    pl.kernel,
    out_type=jax.ShapeDtypeStruct((2, NPAD), jnp.float32),
    mesh=_mesh,
    scratch_types=[
        pltpu.VMEM((CHUNK,), jnp.int32),
        pltpu.VMEM((CHUNK,), jnp.float32),
        pltpu.VMEM_SHARED((NPAD,), jnp.float32),
    ],
)
def _deg_kernel(dst_hbm, zer_hbm, out_hbm, idx_d, ones_v, acc):
    c = lax.axis_index("c")
    s = lax.axis_index("s")
    w = s * 2 + c
    for k in range(CHUNK // 16):
        ones_v[pl.ds(k * 16, 16)] = jnp.ones((16,), jnp.float32)
    pltpu.sync_copy(zer_hbm.at[pl.ds(s * RPS, RPS)], acc.at[pl.ds(s * RPS, RPS)])
    plsc.subcore_barrier()

    def body(i, carry):
        e = (w * NSUP + i) * CHUNK
        pltpu.sync_copy(dst_hbm.at[pl.ds(e, CHUNK)], idx_d)
        pltpu.sync_copy(ones_v, acc.at[idx_d], add=True)
        return carry

    lax.fori_loop(0, NSUP, body, 0)
    plsc.subcore_barrier()
    pltpu.sync_copy(acc.at[pl.ds(s * RPS, RPS)], out_hbm.at[c, pl.ds(s * RPS, RPS)])


@functools.partial(
    pl.kernel,
    out_type=jax.ShapeDtypeStruct((2, NPAD, IN_DIM), jnp.float32),
    mesh=_mesh,
    scratch_types=[
        pltpu.VMEM((CHUNK,), jnp.int32),
        pltpu.VMEM((CHUNK,), jnp.int32),
        pltpu.VMEM((CHUNK,), jnp.int32),
        pltpu.VMEM((CHUNK,), jnp.int32),
        pltpu.VMEM((CHUNK, IN_DIM), jnp.float32),
        pltpu.VMEM((CHUNK, IN_DIM), jnp.float32),
        pltpu.VMEM_SHARED((NPAD, IN_DIM), jnp.float32),
        pltpu.SemaphoreType.DMA,
        pltpu.SemaphoreType.DMA,
    ],
)
def _agg_kernel(feat_hbm, src_hbm, dst_hbm, zer_hbm, out_hbm,
                idx_s0, idx_d0, idx_s1, idx_d1, rows0, rows1, acc, sem0, sem1):
    c = lax.axis_index("c")
    s = lax.axis_index("s")
    w = s * 2 + c
    base = w * NSUP * CHUNK

    pltpu.sync_copy(zer_hbm.at[pl.ds(s * RPS, RPS)], acc.at[pl.ds(s * RPS, RPS)])
    plsc.subcore_barrier()

    pltpu.sync_copy(src_hbm.at[pl.ds(base, CHUNK)], idx_s0)
    pltpu.sync_copy(dst_hbm.at[pl.ds(base, CHUNK)], idx_d0)
    pltpu.async_copy(feat_hbm.at[idx_s0], rows0, sem0)
    pltpu.sync_copy(src_hbm.at[pl.ds(base + CHUNK, CHUNK)], idx_s1)
    pltpu.sync_copy(dst_hbm.at[pl.ds(base + CHUNK, CHUNK)], idx_d1)

    def pair(j, carry):
        pltpu.make_async_copy(feat_hbm.at[idx_s0], rows0, sem0).wait()
        pltpu.async_copy(feat_hbm.at[idx_s1], rows1, sem1)
        pltpu.sync_copy(rows0, acc.at[idx_d0], add=True)

        @pl.when(j < NSUP // 2 - 1)
        def _():
            e = base + (2 * j + 2) * CHUNK
            pltpu.sync_copy(src_hbm.at[pl.ds(e, CHUNK)], idx_s0)
            pltpu.sync_copy(dst_hbm.at[pl.ds(e, CHUNK)], idx_d0)

        pltpu.make_async_copy(feat_hbm.at[idx_s1], rows1, sem1).wait()

        @pl.when(j < NSUP // 2 - 1)
        def _():
            pltpu.async_copy(feat_hbm.at[idx_s0], rows0, sem0)

        pltpu.sync_copy(rows1, acc.at[idx_d1], add=True)

        @pl.when(j < NSUP // 2 - 1)
        def _():
            e = base + (2 * j + 3) * CHUNK
            pltpu.sync_copy(src_hbm.at[pl.ds(e, CHUNK)], idx_s1)
            pltpu.sync_copy(dst_hbm.at[pl.ds(e, CHUNK)], idx_d1)

        return carry

    lax.fori_loop(0, NSUP // 2, pair, 0)
    plsc.subcore_barrier()
    pltpu.sync_copy(acc.at[pl.ds(s * RPS, RPS)],
                    out_hbm.at[c, pl.ds(s * RPS, RPS)])


def _tc1_body(deg_ref, x_ref, y1_ref):
    dinv = lax.rsqrt(deg_ref[0] + deg_ref[1] + 1.0)
    y1_ref[...] = x_ref[...] * dinv


def _tc2_body(deg_ref, agg_ref, y1_ref, w1_ref, b1_ref, wc_ref, y2_ref):
    dinv = lax.rsqrt(deg_ref[0] + deg_ref[1] + 1.0)
    t = (agg_ref[0] + agg_ref[1] + y1_ref[...]) * dinv
    h = jnp.dot(t, w1_ref[...], preferred_element_type=jnp.float32) + b1_ref[...]
    u = jnp.dot(h, wc_ref[...], preferred_element_type=jnp.float32)
    y2_ref[...] = u * dinv


def _tc3_body(deg_ref, agg_ref, y2_ref, bm_ref, blv_ref, noise_ref,
              z_ref, mean_ref, lv_ref):
    dinv = lax.rsqrt(deg_ref[0] + deg_ref[1] + 1.0)
    v = (agg_ref[0] + agg_ref[1] + y2_ref[...]) * dinv
    mean = v[:, :LAT] + bm_ref[...]
    lv = v[:, LAT:] + blv_ref[...]
    mean_ref[...] = mean
    lv_ref[...] = lv
    z_ref[...] = noise_ref[...] * jnp.exp(0.5 * lv) + mean


def _deg_spec():
    return pl.BlockSpec((2, _BLK, 1), lambda i: (0, i, 0))


def _agg_spec():
    return pl.BlockSpec((2, _BLK, IN_DIM), lambda i: (0, i, 0))


def _row_spec(d):
    return pl.BlockSpec((_BLK, d), lambda i: (i, 0))


def _full_spec(r, c):
    return pl.BlockSpec((r, c), lambda i: (0, 0))


def kernel(x, edge_index, W1, b1, Wm, bm, Wlv, blv):
    src = edge_index[0].astype(jnp.int32)
    dst = edge_index[1].astype(jnp.int32)
    npad_e = E_PAD - E
    pad_src = (jnp.arange(npad_e, dtype=jnp.int32) * 37) % N
    pad_dst = N + (jnp.arange(npad_e, dtype=jnp.int32) % (NPAD - N))
    src2 = jnp.concatenate([src, pad_src])
    dst2 = jnp.concatenate([dst, pad_dst])

    zer1 = jnp.zeros((NPAD,), jnp.float32)
    zer2 = jnp.zeros((NPAD, IN_DIM), jnp.float32)
    Wcat = jnp.concatenate([Wm, Wlv], axis=1)
    noise = jax.random.normal(jax.random.key(42), (N, LAT), dtype=x.dtype)

    degp = _deg_kernel(dst2, zer1)
    deg3 = degp.reshape(2, NPAD, 1)

    grid = (N // _BLK,)
    y1 = pl.pallas_call(
        _tc1_body,
        grid=grid,
        in_specs=[_deg_spec(), _row_spec(IN_DIM)],
        out_specs=_row_spec(IN_DIM),
        out_shape=jax.ShapeDtypeStruct((N, IN_DIM), jnp.float32),
    )(deg3, x)

    agg1 = _agg_kernel(y1, src2, dst2, zer2)

    y2 = pl.pallas_call(
        _tc2_body,
        grid=grid,
        in_specs=[_deg_spec(), _agg_spec(), _row_spec(IN_DIM),
                  _full_spec(IN_DIM, HID), _full_spec(1, HID),
                  _full_spec(HID, IN_DIM)],
        out_specs=_row_spec(IN_DIM),
        out_shape=jax.ShapeDtypeStruct((N, IN_DIM), jnp.float32),
    )(deg3, agg1, y1, W1, b1.reshape(1, HID), Wcat)

    agg2 = _agg_kernel(y2, src2, dst2, zer2)

    z, mean, lv = pl.pallas_call(
        _tc3_body,
        grid=grid,
        in_specs=[_deg_spec(), _agg_spec(), _row_spec(IN_DIM),
                  _full_spec(1, LAT), _full_spec(1, LAT), _row_spec(LAT)],
        out_specs=[_row_spec(LAT), _row_spec(LAT), _row_spec(LAT)],
        out_shape=[jax.ShapeDtypeStruct((N, LAT), jnp.float32)] * 3,
    )(deg3, agg2, y2, bm.reshape(1, LAT), blv.reshape(1, LAT), noise)

    return (z, mean, lv)


# speedup vs baseline: 40.6292x; 1.3766x over previous
import functools

import jax
import jax.numpy as jnp
from jax import lax
from jax.experimental import pallas as pl
from jax.experimental.pallas import tpu as pltpu
from jax.experimental.pallas import tpu_sc as plsc

N = 10000
NPAD = 10240
E = 320000
IN_DIM = 128
HID = 256
LAT = 64

CHUNK = 128
NW = 32
NSUP = -(-E // (CHUNK * NW))
NSUP += NSUP % 2
E_PAD = NSUP * CHUNK * NW
RPS = NPAD // 16

_BLK = 1000

_mesh = plsc.VectorSubcoreMesh(core_axis_name="c", subcore_axis_name="s")


@functools.partial(
    pl.kernel,
    out_type=jax.ShapeDtypeStruct((2, NPAD), jnp.float32),
    mesh=_mesh,
    scratch_types=[
        pltpu.VMEM((NSUP, CHUNK), jnp.int32),
        pltpu.VMEM((CHUNK,), jnp.float32),
        pltpu.VMEM_SHARED((NPAD,), jnp.float32),
        pltpu.SemaphoreType.DMA,
    ],
)
def _deg_kernel(dst_hbm, zer_hbm, out_hbm, idx_d, ones_v, acc, sem):
    c = lax.axis_index("c")
    s = lax.axis_index("s")
    w = s * 2 + c
    for k in range(CHUNK // 16):
        ones_v[pl.ds(k * 16, 16)] = jnp.ones((16,), jnp.float32)
    pltpu.sync_copy(dst_hbm.at[pl.ds(w * NSUP, NSUP)], idx_d)
    pltpu.sync_copy(zer_hbm.at[pl.ds(s * RPS, RPS)], acc.at[pl.ds(s * RPS, RPS)])
    plsc.subcore_barrier()

    def body(i, carry):
        pltpu.async_copy(ones_v, acc.at[idx_d.at[i]], sem, add=True)

        @pl.when(i >= 8)
        def _():
            pltpu.make_async_copy(ones_v, acc.at[idx_d.at[i]], sem).wait()

        return carry

    lax.fori_loop(0, NSUP, body, 0)
    for _ in range(8):
        pltpu.make_async_copy(ones_v, acc.at[idx_d.at[0]], sem).wait()
    plsc.subcore_barrier()
    pltpu.sync_copy(acc.at[pl.ds(s * RPS, RPS)], out_hbm.at[c, pl.ds(s * RPS, RPS)])


@functools.partial(
    pl.kernel,
    out_type=jax.ShapeDtypeStruct((2, NPAD, IN_DIM), jnp.float32),
    mesh=_mesh,
    scratch_types=[
        pltpu.VMEM((NSUP, CHUNK), jnp.int32),
        pltpu.VMEM((2, 8, CHUNK), jnp.int32),
        pltpu.VMEM((2, CHUNK, IN_DIM), jnp.float32),
        pltpu.VMEM_SHARED((NPAD, IN_DIM), jnp.float32),
        pltpu.SemaphoreType.DMA,
        pltpu.SemaphoreType.DMA,
        pltpu.SemaphoreType.DMA,
        pltpu.SemaphoreType.DMA,
    ],
)
def _agg_kernel(feat_hbm, src_hbm, dst_hbm, zer_hbm, out_hbm,
                idx_s, idx_d, rows, acc, g0, g1, s0, s1):
    c = lax.axis_index("c")
    ss = lax.axis_index("s")
    w = ss * 2 + c
    gsem = (g0, g1)
    ssem = (s0, s1)
    base = w * NSUP

    pltpu.sync_copy(src_hbm.at[pl.ds(base, NSUP)], idx_s)
    pltpu.sync_copy(dst_hbm.at[pl.ds(base, 8)], idx_d.at[0])
    pltpu.sync_copy(zer_hbm.at[pl.ds(ss * RPS, RPS)], acc.at[pl.ds(ss * RPS, RPS)])
    plsc.subcore_barrier()

    pltpu.async_copy(feat_hbm.at[idx_s.at[0]], rows.at[0], g0)

    def body(i, carry):
        for kk in range(16):
            k = i * 16 + kk
            sl = kk % 2
            osl = (kk + 1) % 2
            p = kk // 8

            @pl.when(k >= 1)
            def _():
                pltpu.make_async_copy(rows.at[osl], acc.at[idx_d.at[p, 0]],
                                      ssem[osl]).wait()

            @pl.when(k + 1 <= NSUP - 1)
            def _():
                pltpu.async_copy(feat_hbm.at[idx_s.at[k + 1]],
                                 rows.at[osl], gsem[osl])

            pltpu.make_async_copy(feat_hbm.at[idx_s.at[k]],
                                  rows.at[sl], gsem[sl]).wait()
            pltpu.async_copy(rows.at[sl], acc.at[idx_d.at[p, kk % 8]],
                             ssem[sl], add=True)

            if kk == 0:
                pltpu.sync_copy(dst_hbm.at[pl.ds(base + i * 16 + 8, 8)],
                                idx_d.at[1])
            if kk == 8:
                @pl.when(i < NSUP // 16 - 1)
                def _():
                    pltpu.sync_copy(dst_hbm.at[pl.ds(base + i * 16 + 16, 8)],
                                    idx_d.at[0])
        return carry

    lax.fori_loop(0, NSUP // 16, body, 0)
    pltpu.make_async_copy(rows.at[1], acc.at[idx_d.at[0, 0]], ssem[1]).wait()
    plsc.subcore_barrier()
    pltpu.sync_copy(acc.at[pl.ds(ss * RPS, RPS)],
                    out_hbm.at[c, pl.ds(ss * RPS, RPS)])


def _tc1_body(deg_ref, x_ref, y1_ref):
    dinv = lax.rsqrt(deg_ref[0] + deg_ref[1] + 1.0)
    y1_ref[...] = x_ref[...] * dinv


def _tc2_body(deg_ref, agg_ref, y1_ref, w1_ref, b1_ref, wc_ref, y2_ref):
    dinv = lax.rsqrt(deg_ref[0] + deg_ref[1] + 1.0)
    t = (agg_ref[0] + agg_ref[1] + y1_ref[...]) * dinv
    h = jnp.dot(t, w1_ref[...], preferred_element_type=jnp.float32) + b1_ref[...]
    u = jnp.dot(h, wc_ref[...], preferred_element_type=jnp.float32)
    y2_ref[...] = u * dinv


def _tc3_body(deg_ref, agg_ref, y2_ref, bm_ref, blv_ref, noise_ref,
              z_ref, mean_ref, lv_ref):
    dinv = lax.rsqrt(deg_ref[0] + deg_ref[1] + 1.0)
    v = (agg_ref[0] + agg_ref[1] + y2_ref[...]) * dinv
    mean = v[:, :LAT] + bm_ref[...]
    lv = v[:, LAT:] + blv_ref[...]
    mean_ref[...] = mean
    lv_ref[...] = lv
    z_ref[...] = noise_ref[...] * jnp.exp(0.5 * lv) + mean


def _deg_spec():
    return pl.BlockSpec((2, _BLK, 1), lambda i: (0, i, 0))


def _agg_spec():
    return pl.BlockSpec((2, _BLK, IN_DIM), lambda i: (0, i, 0))


def _row_spec(d):
    return pl.BlockSpec((_BLK, d), lambda i: (i, 0))


def _full_spec(r, c):
    return pl.BlockSpec((r, c), lambda i: (0, 0))


def kernel(x, edge_index, W1, b1, Wm, bm, Wlv, blv):
    src = edge_index[0].astype(jnp.int32)
    dst = edge_index[1].astype(jnp.int32)
    npad_e = E_PAD - E
    pad_src = (jnp.arange(npad_e, dtype=jnp.int32) * 37) % N
    pad_dst = N + (jnp.arange(npad_e, dtype=jnp.int32) % (NPAD - N))
    src2 = jnp.concatenate([src, pad_src]).reshape(E_PAD // CHUNK, CHUNK)
    dst2 = jnp.concatenate([dst, pad_dst]).reshape(E_PAD // CHUNK, CHUNK)

    zer1 = jnp.zeros((NPAD,), jnp.float32)
    zer2 = jnp.zeros((NPAD, IN_DIM), jnp.float32)
    Wcat = jnp.concatenate([Wm, Wlv], axis=1)
    noise = jax.random.normal(jax.random.key(42), (N, LAT), dtype=x.dtype)

    degp = _deg_kernel(dst2, zer1)
    deg3 = degp.reshape(2, NPAD, 1)

    grid = (N // _BLK,)
    y1 = pl.pallas_call(
        _tc1_body,
        grid=grid,
        in_specs=[_deg_spec(), _row_spec(IN_DIM)],
        out_specs=_row_spec(IN_DIM),
        out_shape=jax.ShapeDtypeStruct((N, IN_DIM), jnp.float32),
    )(deg3, x)

    agg1 = _agg_kernel(y1, src2, dst2, zer2)

    y2 = pl.pallas_call(
        _tc2_body,
        grid=grid,
        in_specs=[_deg_spec(), _agg_spec(), _row_spec(IN_DIM),
                  _full_spec(IN_DIM, HID), _full_spec(1, HID),
                  _full_spec(HID, IN_DIM)],
        out_specs=_row_spec(IN_DIM),
        out_shape=jax.ShapeDtypeStruct((N, IN_DIM), jnp.float32),
    )(deg3, agg1, y1, W1, b1.reshape(1, HID), Wcat)

    agg2 = _agg_kernel(y2, src2, dst2, zer2)

    z, mean, lv = pl.pallas_call(
        _tc3_body,
        grid=grid,
        in_specs=[_deg_spec(), _agg_spec(), _row_spec(IN_DIM),
                  _full_spec(1, LAT), _full_spec(1, LAT), _row_spec(LAT)],
        out_specs=[_row_spec(LAT), _row_spec(LAT), _row_spec(LAT)],
        out_shape=[jax.ShapeDtypeStruct((N, LAT), jnp.float32)] * 3,
    )(deg3, agg2, y2, bm.reshape(1, LAT), blv.reshape(1, LAT), noise)

    return (z, mean, lv)
